# baseline (speedup 1.0000x reference)
import jax
import jax.numpy as jnp
from jax import lax
from jax.experimental import pallas as pl
from jax.experimental.pallas import tpu as pltpu, tpu_sc as plsc

NC, NS = 2, 16
NW = NC * NS
RC = 16
LANES = 16


def _sc_body(x_hbm, pe_hbm, out_hbm, bx, bp, sin, sout, spe):
    B = 4
    L = pe_hbm.shape[0]
    D = pe_hbm.shape[1]
    lw = L // NW
    nt = lw // RC
    ns = nt * B

    wid = lax.axis_index("s") * NC + lax.axis_index("c")
    l0 = wid * lw

    def pe_start(t):
        pltpu.async_copy(pe_hbm.at[pl.ds(l0 + t * RC, RC)], bp[t % 2],
                         spe[t % 2])

    def pe_wait(t):
        pltpu.make_async_copy(pe_hbm.at[pl.ds(l0 + t * RC, RC)], bp[t % 2],
                              spe[t % 2]).wait()

    def row0(s):
        t, b = divmod(s, B)
        return b * L + l0 + t * RC

    def in_start(s):
        pltpu.async_copy(x_hbm.at[pl.ds(row0(s), RC)], bx[s % 5], sin[s % 5])

    def in_wait(s):
        pltpu.make_async_copy(x_hbm.at[pl.ds(row0(s), RC)], bx[s % 5],
                              sin[s % 5]).wait()

    def out_start(s):
        pltpu.async_copy(bx[s % 5], out_hbm.at[pl.ds(row0(s), RC)],
                         sout[s % 5])

    def out_wait(s):
        pltpu.make_async_copy(bx[s % 5], out_hbm.at[pl.ds(row0(s), RC)],
                              sout[s % 5]).wait()

    pe_start(0)
    in_start(0)
    in_start(1)
    in_start(2)

    for s in range(ns):
        t, b = divmod(s, B)
        if b == 0:
            if t + 1 < nt:
                pe_start(t + 1)
            pe_wait(t)
        if s >= 2:
            out_wait(s - 2)
        if s + 3 < ns:
            in_start(s + 3)
        in_wait(s)

        bxs = bx[s % 5]
        bps = bp[t % 2]

        @plsc.parallel_loop(0, RC * (D // LANES), unroll=8)
        def add(k):
            r = k >> 6
            c = (k & 63) * LANES
            plsc.addupdate(bxs.at[r, pl.ds(c, LANES)], bps[r, pl.ds(c, LANES)])

        out_start(s)

    out_wait(ns - 2)
    out_wait(ns - 1)


def kernel(x, pe):
    B, L, D = x.shape
    run = pl.kernel(
        _sc_body,
        out_type=jax.ShapeDtypeStruct((B * L, D), x.dtype),
        mesh=plsc.VectorSubcoreMesh(core_axis_name="c", subcore_axis_name="s"),
        scratch_types=[
            [pltpu.VMEM((RC, D), jnp.float32) for _ in range(5)],
            [pltpu.VMEM((RC, D), jnp.float32) for _ in range(2)],
            [pltpu.SemaphoreType.DMA for _ in range(5)],
            [pltpu.SemaphoreType.DMA for _ in range(5)],
            [pltpu.SemaphoreType.DMA for _ in range(2)],
        ],
    )
    return run(x.reshape(B * L, D), pe).reshape(B, L, D)


# speedup vs baseline: 1.0616x; 1.0616x over previous
import jax
import jax.numpy as jnp
from jax import lax
from jax.experimental import pallas as pl
from jax.experimental.pallas import tpu as pltpu, tpu_sc as plsc

NC, NS = 2, 16
NW = NC * NS
LS = 7168
L_BLK = 1792
RC = 16
LANES = 16


def _tc_body(x_ref, pe_ref, o_ref):
    o_ref[0] = x_ref[0] + pe_ref[...]


def _sc_body(x_hbm, pe_hbm, out_hbm, bx, bp, sin, sout, spe):
    B = 4
    L = pe_hbm.shape[0]
    D = pe_hbm.shape[1]
    rows_sc = L - LS
    lw = rows_sc // NW
    nt = lw // RC
    ns = nt * B

    wid = lax.axis_index("s") * NC + lax.axis_index("c")
    l0 = LS + wid * lw

    def pe_start(t):
        pltpu.async_copy(pe_hbm.at[pl.ds(l0 + t * RC, RC)], bp[t % 2],
                         spe[t % 2])

    def pe_wait(t):
        pltpu.make_async_copy(pe_hbm.at[pl.ds(l0 + t * RC, RC)], bp[t % 2],
                              spe[t % 2]).wait()

    def xrow0(s):
        t, b = divmod(s, B)
        return b * L + l0 + t * RC

    def orow0(s):
        t, b = divmod(s, B)
        return b * rows_sc + wid * lw + t * RC

    def in_start(s):
        pltpu.async_copy(x_hbm.at[pl.ds(xrow0(s), RC)], bx[s % 5], sin[s % 5])

    def in_wait(s):
        pltpu.make_async_copy(x_hbm.at[pl.ds(xrow0(s), RC)], bx[s % 5],
                              sin[s % 5]).wait()

    def out_start(s):
        pltpu.async_copy(bx[s % 5], out_hbm.at[pl.ds(orow0(s), RC)],
                         sout[s % 5])

    def out_wait(s):
        pltpu.make_async_copy(bx[s % 5], out_hbm.at[pl.ds(orow0(s), RC)],
                              sout[s % 5]).wait()

    pe_start(0)
    in_start(0)
    in_start(1)
    in_start(2)

    for s in range(ns):
        t, b = divmod(s, B)
        if b == 0:
            if t + 1 < nt:
                pe_start(t + 1)
            pe_wait(t)
        if s >= 2:
            out_wait(s - 2)
        if s + 3 < ns:
            in_start(s + 3)
        in_wait(s)

        bxs = bx[s % 5]
        bps = bp[t % 2]

        @plsc.parallel_loop(0, RC * (D // LANES), unroll=8)
        def add(k):
            r = k >> 6
            c = (k & 63) * LANES
            plsc.addupdate(bxs.at[r, pl.ds(c, LANES)], bps[r, pl.ds(c, LANES)])

        out_start(s)

    out_wait(ns - 2)
    out_wait(ns - 1)


def kernel(x, pe):
    B, L, D = x.shape

    tc_out = pl.pallas_call(
        _tc_body,
        grid=(LS // L_BLK, B),
        in_specs=[
            pl.BlockSpec((1, L_BLK, D), lambda i, b: (b, i, 0)),
            pl.BlockSpec((L_BLK, D), lambda i, b: (i, 0)),
        ],
        out_specs=pl.BlockSpec((1, L_BLK, D), lambda i, b: (b, i, 0)),
        out_shape=jax.ShapeDtypeStruct((B, L, D), x.dtype),
    )(x, pe)

    rows_sc = L - LS
    sc_run = pl.kernel(
        _sc_body,
        out_type=jax.ShapeDtypeStruct((B * rows_sc, D), x.dtype),
        mesh=plsc.VectorSubcoreMesh(core_axis_name="c", subcore_axis_name="s"),
        scratch_types=[
            [pltpu.VMEM((RC, D), jnp.float32) for _ in range(5)],
            [pltpu.VMEM((RC, D), jnp.float32) for _ in range(2)],
            [pltpu.SemaphoreType.DMA for _ in range(5)],
            [pltpu.SemaphoreType.DMA for _ in range(5)],
            [pltpu.SemaphoreType.DMA for _ in range(2)],
        ],
    )
    sc_out = sc_run(x.reshape(B * L, D), pe)

    return lax.dynamic_update_slice(
        tc_out, sc_out.reshape(B, rows_sc, D), (0, LS, 0)
    )


# speedup vs baseline: 1.1077x; 1.0434x over previous
import jax
import jax.numpy as jnp
from jax import lax
from jax.experimental import pallas as pl
from jax.experimental.pallas import tpu as pltpu, tpu_sc as plsc

NC, NS = 2, 16
NW = NC * NS
LS = 7680
L_BLK = 1920
RC = 16
LANES = 16


def _tc_body(x_ref, pe_ref, o_ref):
    o_ref[0] = x_ref[0] + pe_ref[...]


def _sc_body(x_hbm, pe_hbm, out_hbm, bx, bp, sin, sout, spe):
    B = 4
    L = pe_hbm.shape[0]
    D = pe_hbm.shape[1]
    rows_sc = L - LS
    lw = rows_sc // NW
    nt = lw // RC
    ns = nt * B

    wid = lax.axis_index("s") * NC + lax.axis_index("c")
    l0 = LS + wid * lw

    def pe_start(t):
        pltpu.async_copy(pe_hbm.at[pl.ds(l0 + t * RC, RC)], bp[t % 2],
                         spe[t % 2])

    def pe_wait(t):
        pltpu.make_async_copy(pe_hbm.at[pl.ds(l0 + t * RC, RC)], bp[t % 2],
                              spe[t % 2]).wait()

    def xrow0(s):
        t, b = divmod(s, B)
        return b * L + l0 + t * RC

    def orow0(s):
        t, b = divmod(s, B)
        return b * rows_sc + wid * lw + t * RC

    def in_start(s):
        pltpu.async_copy(x_hbm.at[pl.ds(xrow0(s), RC)], bx[s % 5], sin[s % 5])

    def in_wait(s):
        pltpu.make_async_copy(x_hbm.at[pl.ds(xrow0(s), RC)], bx[s % 5],
                              sin[s % 5]).wait()

    def out_start(s):
        pltpu.async_copy(bx[s % 5], out_hbm.at[pl.ds(orow0(s), RC)],
                         sout[s % 5])

    def out_wait(s):
        pltpu.make_async_copy(bx[s % 5], out_hbm.at[pl.ds(orow0(s), RC)],
                              sout[s % 5]).wait()

    pe_start(0)
    in_start(0)
    in_start(1)
    in_start(2)

    for s in range(ns):
        t, b = divmod(s, B)
        if b == 0:
            if t + 1 < nt:
                pe_start(t + 1)
            pe_wait(t)
        if s >= 2:
            out_wait(s - 2)
        if s + 3 < ns:
            in_start(s + 3)
        in_wait(s)

        bxs = bx[s % 5]
        bps = bp[t % 2]

        @plsc.parallel_loop(0, RC * (D // LANES), unroll=8)
        def add(k):
            r = k >> 6
            c = (k & 63) * LANES
            plsc.addupdate(bxs.at[r, pl.ds(c, LANES)], bps[r, pl.ds(c, LANES)])

        out_start(s)

    out_wait(ns - 2)
    out_wait(ns - 1)


def kernel(x, pe):
    B, L, D = x.shape

    tc_out = pl.pallas_call(
        _tc_body,
        grid=(LS // L_BLK, B),
        in_specs=[
            pl.BlockSpec((1, L_BLK, D), lambda i, b: (b, i, 0)),
            pl.BlockSpec((L_BLK, D), lambda i, b: (i, 0)),
        ],
        out_specs=pl.BlockSpec((1, L_BLK, D), lambda i, b: (b, i, 0)),
        out_shape=jax.ShapeDtypeStruct((B, L, D), x.dtype),
    )(x, pe)

    rows_sc = L - LS
    sc_run = pl.kernel(
        _sc_body,
        out_type=jax.ShapeDtypeStruct((B * rows_sc, D), x.dtype),
        mesh=plsc.VectorSubcoreMesh(core_axis_name="c", subcore_axis_name="s"),
        scratch_types=[
            [pltpu.VMEM((RC, D), jnp.float32) for _ in range(5)],
            [pltpu.VMEM((RC, D), jnp.float32) for _ in range(2)],
            [pltpu.SemaphoreType.DMA for _ in range(5)],
            [pltpu.SemaphoreType.DMA for _ in range(5)],
            [pltpu.SemaphoreType.DMA for _ in range(2)],
        ],
    )
    sc_out = sc_run(x.reshape(B * L, D), pe)

    return lax.dynamic_update_slice(
        tc_out, sc_out.reshape(B, rows_sc, D), (0, LS, 0)
    )


# speedup vs baseline: 1.1386x; 1.0279x over previous
import jax
import jax.numpy as jnp
from jax import lax
from jax.experimental import pallas as pl
from jax.experimental.pallas import tpu as pltpu, tpu_sc as plsc

NC, NS = 2, 16
NW = NC * NS
LS = 7936
L_BLK = 1984
RC = 8
LANES = 16


def _tc_body(x_ref, pe_ref, o_ref):
    o_ref[0] = x_ref[0] + pe_ref[...]


def _sc_body(x_hbm, pe_hbm, out_hbm, bx, bp, sin, sout, spe):
    B = 4
    L = pe_hbm.shape[0]
    D = pe_hbm.shape[1]
    rows_sc = L - LS
    lw = rows_sc // NW
    nt = lw // RC
    ns = nt * B

    wid = lax.axis_index("s") * NC + lax.axis_index("c")
    l0 = LS + wid * lw

    def pe_start(t):
        pltpu.async_copy(pe_hbm.at[pl.ds(l0 + t * RC, RC)], bp[t % 2],
                         spe[t % 2])

    def pe_wait(t):
        pltpu.make_async_copy(pe_hbm.at[pl.ds(l0 + t * RC, RC)], bp[t % 2],
                              spe[t % 2]).wait()

    def xrow0(s):
        t, b = divmod(s, B)
        return b * L + l0 + t * RC

    def orow0(s):
        t, b = divmod(s, B)
        return b * rows_sc + wid * lw + t * RC

    def in_start(s):
        pltpu.async_copy(x_hbm.at[pl.ds(xrow0(s), RC)], bx[s % 5], sin[s % 5])

    def in_wait(s):
        pltpu.make_async_copy(x_hbm.at[pl.ds(xrow0(s), RC)], bx[s % 5],
                              sin[s % 5]).wait()

    def out_start(s):
        pltpu.async_copy(bx[s % 5], out_hbm.at[pl.ds(orow0(s), RC)],
                         sout[s % 5])

    def out_wait(s):
        pltpu.make_async_copy(bx[s % 5], out_hbm.at[pl.ds(orow0(s), RC)],
                              sout[s % 5]).wait()

    pe_start(0)
    in_start(0)
    in_start(1)
    in_start(2)

    for s in range(ns):
        t, b = divmod(s, B)
        if b == 0:
            if t + 1 < nt:
                pe_start(t + 1)
            pe_wait(t)
        if s >= 2:
            out_wait(s - 2)
        if s + 3 < ns:
            in_start(s + 3)
        in_wait(s)

        bxs = bx[s % 5]
        bps = bp[t % 2]

        @plsc.parallel_loop(0, RC * (D // LANES), unroll=8)
        def add(k):
            r = k >> 6
            c = (k & 63) * LANES
            plsc.addupdate(bxs.at[r, pl.ds(c, LANES)], bps[r, pl.ds(c, LANES)])

        out_start(s)

    out_wait(ns - 2)
    out_wait(ns - 1)


def kernel(x, pe):
    B, L, D = x.shape

    tc_out = pl.pallas_call(
        _tc_body,
        grid=(LS // L_BLK, B),
        in_specs=[
            pl.BlockSpec((1, L_BLK, D), lambda i, b: (b, i, 0)),
            pl.BlockSpec((L_BLK, D), lambda i, b: (i, 0)),
        ],
        out_specs=pl.BlockSpec((1, L_BLK, D), lambda i, b: (b, i, 0)),
        out_shape=jax.ShapeDtypeStruct((B, L, D), x.dtype),
    )(x, pe)

    rows_sc = L - LS
    sc_run = pl.kernel(
        _sc_body,
        out_type=jax.ShapeDtypeStruct((B * rows_sc, D), x.dtype),
        mesh=plsc.VectorSubcoreMesh(core_axis_name="c", subcore_axis_name="s"),
        scratch_types=[
            [pltpu.VMEM((RC, D), jnp.float32) for _ in range(5)],
            [pltpu.VMEM((RC, D), jnp.float32) for _ in range(2)],
            [pltpu.SemaphoreType.DMA for _ in range(5)],
            [pltpu.SemaphoreType.DMA for _ in range(5)],
            [pltpu.SemaphoreType.DMA for _ in range(2)],
        ],
    )
    sc_out = sc_run(x.reshape(B * L, D), pe)

    return lax.dynamic_update_slice(
        tc_out, sc_out.reshape(B, rows_sc, D), (0, LS, 0)
    )


# speedup vs baseline: 1.1613x; 1.0199x over previous
import jax
import jax.numpy as jnp
from jax import lax
from jax.experimental import pallas as pl
from jax.experimental.pallas import tpu as pltpu, tpu_sc as plsc

NC, NS = 2, 16
NW = NC * NS
LS = 8064
L_BLK = 2016
RC = 4
LANES = 16


def _tc_body(x_ref, pe_ref, o_ref):
    o_ref[0] = x_ref[0] + pe_ref[...]


def _sc_body(x_hbm, pe_hbm, out_hbm, bx, bp, sin, sout, spe):
    B = 4
    L = pe_hbm.shape[0]
    D = pe_hbm.shape[1]
    rows_sc = L - LS
    lw = rows_sc // NW
    nt = lw // RC
    ns = nt * B

    wid = lax.axis_index("s") * NC + lax.axis_index("c")
    l0 = LS + wid * lw

    def pe_start(t):
        pltpu.async_copy(pe_hbm.at[pl.ds(l0 + t * RC, RC)], bp[t % 2],
                         spe[t % 2])

    def pe_wait(t):
        pltpu.make_async_copy(pe_hbm.at[pl.ds(l0 + t * RC, RC)], bp[t % 2],
                              spe[t % 2]).wait()

    def xrow0(s):
        t, b = divmod(s, B)
        return b * L + l0 + t * RC

    def orow0(s):
        t, b = divmod(s, B)
        return b * rows_sc + wid * lw + t * RC

    def in_start(s):
        pltpu.async_copy(x_hbm.at[pl.ds(xrow0(s), RC)], bx[s % 5], sin[s % 5])

    def in_wait(s):
        pltpu.make_async_copy(x_hbm.at[pl.ds(xrow0(s), RC)], bx[s % 5],
                              sin[s % 5]).wait()

    def out_start(s):
        pltpu.async_copy(bx[s % 5], out_hbm.at[pl.ds(orow0(s), RC)],
                         sout[s % 5])

    def out_wait(s):
        pltpu.make_async_copy(bx[s % 5], out_hbm.at[pl.ds(orow0(s), RC)],
                              sout[s % 5]).wait()

    pe_start(0)
    in_start(0)
    in_start(1)
    in_start(2)

    for s in range(ns):
        t, b = divmod(s, B)
        if b == 0:
            if t + 1 < nt:
                pe_start(t + 1)
            pe_wait(t)
        if s >= 2:
            out_wait(s - 2)
        if s + 3 < ns:
            in_start(s + 3)
        in_wait(s)

        bxs = bx[s % 5]
        bps = bp[t % 2]

        @plsc.parallel_loop(0, RC * (D // LANES), unroll=8)
        def add(k):
            r = k >> 6
            c = (k & 63) * LANES
            plsc.addupdate(bxs.at[r, pl.ds(c, LANES)], bps[r, pl.ds(c, LANES)])

        out_start(s)

    out_wait(ns - 2)
    out_wait(ns - 1)


def kernel(x, pe):
    B, L, D = x.shape

    tc_out = pl.pallas_call(
        _tc_body,
        grid=(LS // L_BLK, B),
        in_specs=[
            pl.BlockSpec((1, L_BLK, D), lambda i, b: (b, i, 0)),
            pl.BlockSpec((L_BLK, D), lambda i, b: (i, 0)),
        ],
        out_specs=pl.BlockSpec((1, L_BLK, D), lambda i, b: (b, i, 0)),
        out_shape=jax.ShapeDtypeStruct((B, L, D), x.dtype),
    )(x, pe)

    rows_sc = L - LS
    sc_run = pl.kernel(
        _sc_body,
        out_type=jax.ShapeDtypeStruct((B * rows_sc, D), x.dtype),
        mesh=plsc.VectorSubcoreMesh(core_axis_name="c", subcore_axis_name="s"),
        scratch_types=[
            [pltpu.VMEM((RC, D), jnp.float32) for _ in range(5)],
            [pltpu.VMEM((RC, D), jnp.float32) for _ in range(2)],
            [pltpu.SemaphoreType.DMA for _ in range(5)],
            [pltpu.SemaphoreType.DMA for _ in range(5)],
            [pltpu.SemaphoreType.DMA for _ in range(2)],
        ],
    )
    sc_out = sc_run(x.reshape(B * L, D), pe)

    return lax.dynamic_update_slice(
        tc_out, sc_out.reshape(B, rows_sc, D), (0, LS, 0)
    )


# speedup vs baseline: 1.1720x; 1.0092x over previous
import jax
import jax.numpy as jnp
from jax import lax
from jax.experimental import pallas as pl
from jax.experimental.pallas import tpu as pltpu, tpu_sc as plsc

NC, NS = 2, 16
NW = NC * NS
LS = 8128
L_BLK = 2032
RC = 2
LANES = 16


def _tc_body(x_ref, pe_ref, o_ref):
    o_ref[0] = x_ref[0] + pe_ref[...]


def _sc_body(x_hbm, pe_hbm, out_hbm, bx, bp, sin, sout, spe):
    B = 4
    L = pe_hbm.shape[0]
    D = pe_hbm.shape[1]
    rows_sc = L - LS
    lw = rows_sc // NW
    nt = lw // RC
    ns = nt * B

    wid = lax.axis_index("s") * NC + lax.axis_index("c")
    l0 = LS + wid * lw

    def pe_start(t):
        pltpu.async_copy(pe_hbm.at[pl.ds(l0 + t * RC, RC)], bp[t % 2],
                         spe[t % 2])

    def pe_wait(t):
        pltpu.make_async_copy(pe_hbm.at[pl.ds(l0 + t * RC, RC)], bp[t % 2],
                              spe[t % 2]).wait()

    def xrow0(s):
        t, b = divmod(s, B)
        return b * L + l0 + t * RC

    def orow0(s):
        t, b = divmod(s, B)
        return b * rows_sc + wid * lw + t * RC

    def in_start(s):
        pltpu.async_copy(x_hbm.at[pl.ds(xrow0(s), RC)], bx[s % 5], sin[s % 5])

    def in_wait(s):
        pltpu.make_async_copy(x_hbm.at[pl.ds(xrow0(s), RC)], bx[s % 5],
                              sin[s % 5]).wait()

    def out_start(s):
        pltpu.async_copy(bx[s % 5], out_hbm.at[pl.ds(orow0(s), RC)],
                         sout[s % 5])

    def out_wait(s):
        pltpu.make_async_copy(bx[s % 5], out_hbm.at[pl.ds(orow0(s), RC)],
                              sout[s % 5]).wait()

    pe_start(0)
    in_start(0)
    in_start(1)
    in_start(2)

    for s in range(ns):
        t, b = divmod(s, B)
        if b == 0:
            if t + 1 < nt:
                pe_start(t + 1)
            pe_wait(t)
        if s >= 2:
            out_wait(s - 2)
        if s + 3 < ns:
            in_start(s + 3)
        in_wait(s)

        bxs = bx[s % 5]
        bps = bp[t % 2]

        @plsc.parallel_loop(0, RC * (D // LANES), unroll=8)
        def add(k):
            r = k >> 6
            c = (k & 63) * LANES
            plsc.addupdate(bxs.at[r, pl.ds(c, LANES)], bps[r, pl.ds(c, LANES)])

        out_start(s)

    out_wait(ns - 2)
    out_wait(ns - 1)


def kernel(x, pe):
    B, L, D = x.shape

    tc_out = pl.pallas_call(
        _tc_body,
        grid=(LS // L_BLK, B),
        in_specs=[
            pl.BlockSpec((1, L_BLK, D), lambda i, b: (b, i, 0)),
            pl.BlockSpec((L_BLK, D), lambda i, b: (i, 0)),
        ],
        out_specs=pl.BlockSpec((1, L_BLK, D), lambda i, b: (b, i, 0)),
        out_shape=jax.ShapeDtypeStruct((B, L, D), x.dtype),
    )(x, pe)

    rows_sc = L - LS
    sc_run = pl.kernel(
        _sc_body,
        out_type=jax.ShapeDtypeStruct((B * rows_sc, D), x.dtype),
        mesh=plsc.VectorSubcoreMesh(core_axis_name="c", subcore_axis_name="s"),
        scratch_types=[
            [pltpu.VMEM((RC, D), jnp.float32) for _ in range(5)],
            [pltpu.VMEM((RC, D), jnp.float32) for _ in range(2)],
            [pltpu.SemaphoreType.DMA for _ in range(5)],
            [pltpu.SemaphoreType.DMA for _ in range(5)],
            [pltpu.SemaphoreType.DMA for _ in range(2)],
        ],
    )
    sc_out = sc_run(x.reshape(B * L, D), pe)

    return lax.dynamic_update_slice(
        tc_out, sc_out.reshape(B, rows_sc, D), (0, LS, 0)
    )
